# Initial kernel scaffold; baseline (speedup 1.0000x reference)
import jax
import jax.numpy as jnp
from jax.experimental import pallas as pl


def kernel(coords, pairs, box, A, B, cutoff, atom_types):
    raise NotImplementedError("write your pallas kernel here")


# speedup vs baseline: 203.8508x; 203.8508x over previous
import functools

import jax
import jax.numpy as jnp
from jax import lax
from jax.experimental import pallas as pl
from jax.experimental.pallas import tpu as pltpu
from jax.experimental.pallas import tpu_sc as plsc

NW = 32
CH = 4096
GB = 128
NG = CH // GB
LANES = 16


def _sc_body(ppw, nch, pairs_i_hbm, pairs_j_hbm, packed_hbm, af_hbm, bf_hbm,
             consts_hbm, out_hbm, idx_i_v, idx_j_v, rows_i_v, rows_j_v,
             ene_v, consts_v, a_v, b_v, sem_i, sem_j):
    wid = lax.axis_index("s") * 2 + lax.axis_index("c")
    base = wid * ppw

    pltpu.sync_copy(consts_hbm, consts_v)
    pltpu.sync_copy(af_hbm, a_v)
    pltpu.sync_copy(bf_hbm, b_v)

    ibx = consts_v[pl.ds(0, LANES)]
    iby = consts_v[pl.ds(16, LANES)]
    ibz = consts_v[pl.ds(32, LANES)]
    lbx = consts_v[pl.ds(48, LANES)]
    lby = consts_v[pl.ds(64, LANES)]
    lbz = consts_v[pl.ds(80, LANES)]
    cut = consts_v[pl.ds(96, LANES)]
    zero = jnp.zeros((LANES,), jnp.float32)
    half = zero + 0.5
    one = zero + 1.0

    iota = lax.iota(jnp.int32, LANES)
    col0 = jnp.zeros((LANES,), jnp.int32)
    col1 = col0 + 1
    col2 = col0 + 2
    col3 = col0 + 3

    def bf16r(v):
        b = plsc.bitcast(v, jnp.int32)
        lsb = lax.shift_right_logical(b, 16) & 1
        b = b + lsb + jnp.int32(0x7FFF)
        b = b & jnp.int32(-65536)
        return plsc.bitcast(b, jnp.float32)

    def wrap(d, ib, lb):
        f = bf16r(d) * ib
        f = f + jnp.where(f > half, -one, jnp.where(f < -half, one, zero))
        return bf16r(f) * lb

    @pl.loop(0, nch)
    def _chunk(c):
        off = base + c * CH
        row_off = off // GB
        pltpu.sync_copy(pairs_i_hbm.at[pl.ds(row_off, NG)], idx_i_v)
        pltpu.sync_copy(pairs_j_hbm.at[pl.ds(row_off, NG)], idx_j_v)
        copies = []
        for g in range(NG):
            copies.append(pltpu.async_copy(
                packed_hbm.at[idx_i_v.at[g]],
                rows_i_v.at[pl.ds(g * GB, GB)], sem_i))
            copies.append(pltpu.async_copy(
                packed_hbm.at[idx_j_v.at[g]],
                rows_j_v.at[pl.ds(g * GB, GB)], sem_j))
        for d in copies:
            d.wait()

        @pl.loop(0, CH // LANES)
        def _step(k):
            s = k * LANES
            ridx = iota + s
            xi = plsc.load_gather(rows_i_v, [ridx, col0])
            yi = plsc.load_gather(rows_i_v, [ridx, col1])
            zi = plsc.load_gather(rows_i_v, [ridx, col2])
            ti = plsc.load_gather(rows_i_v, [ridx, col3])
            xj = plsc.load_gather(rows_j_v, [ridx, col0])
            yj = plsc.load_gather(rows_j_v, [ridx, col1])
            zj = plsc.load_gather(rows_j_v, [ridx, col2])
            tj = plsc.load_gather(rows_j_v, [ridx, col3])

            dx = wrap(xj - xi, ibx, lbx)
            dy = wrap(yj - yi, iby, lby)
            dz = wrap(zj - zi, ibz, lbz)
            d2 = dx * dx + dy * dy + dz * dz

            bits = plsc.bitcast(d2, jnp.int32)
            seed = jnp.int32(0x5F3759DF) - lax.shift_right_logical(bits, 1)
            y = plsc.bitcast(seed, jnp.float32)
            hd = 0.5 * d2
            y = y * (1.5 - hd * y * y)
            y = y * (1.5 - hd * y * y)
            y = y * (1.5 - hd * y * y)
            r = d2 * y

            tcomb = ti.astype(jnp.int32) * 16 + tj.astype(jnp.int32)
            a = plsc.load_gather(a_v, [tcomb])
            b = plsc.load_gather(b_v, [tcomb])

            x = b * r
            poly = x * x * (1.0 / 3.0) + x + 1.0
            e = a * poly * jnp.exp(-x)
            e = jnp.where(r <= cut, e, zero)
            ene_v[pl.ds(s, LANES)] = e

        pltpu.sync_copy(ene_v, out_hbm.at[pl.ds(off, CH)])


def kernel(coords, pairs, box, A, B, cutoff, atom_types):
    n = coords.shape[0]
    p = pairs.shape[0]
    nch = -(-p // (NW * CH))
    ppw = nch * CH
    p_pad = NW * ppw

    pairs_i = pairs[:, 0]
    pairs_j = pairs[:, 1]
    pad = p_pad - p
    if pad:
        zpad = jnp.zeros((pad,), jnp.int32)
        pairs_i = jnp.concatenate([pairs_i, zpad])
        pairs_j = jnp.concatenate([pairs_j, zpad])
    pairs_i = pairs_i.reshape(p_pad // GB, GB)
    pairs_j = pairs_j.reshape(p_pad // GB, GB)

    packed = jnp.concatenate(
        [coords.astype(jnp.float32),
         atom_types.astype(jnp.float32).reshape(n, 1),
         jnp.zeros((n, 4), jnp.float32)], axis=1)
    a_flat = A.astype(jnp.float32).reshape(-1)
    b_flat = B.astype(jnp.float32).reshape(-1)

    inv_box = jnp.linalg.inv(box)
    ib = jnp.diagonal(inv_box).astype(jnp.bfloat16).astype(jnp.float32)
    lb = jnp.diagonal(box).astype(jnp.bfloat16).astype(jnp.float32)
    cutf = jnp.asarray(cutoff, jnp.float32)
    vals = jnp.stack([ib[0], ib[1], ib[2], lb[0], lb[1], lb[2],
                      cutf, jnp.float32(0.0)])
    consts = jnp.repeat(vals, LANES)

    mesh = plsc.VectorSubcoreMesh(core_axis_name="c", subcore_axis_name="s")
    run = pl.kernel(
        functools.partial(_sc_body, ppw, nch),
        out_type=jax.ShapeDtypeStruct((p_pad,), jnp.float32),
        mesh=mesh,
        compiler_params=pltpu.CompilerParams(
            needs_layout_passes=False, use_tc_tiling_on_sc=False),
        scratch_types=[
            pltpu.VMEM((NG, GB), jnp.int32),
            pltpu.VMEM((NG, GB), jnp.int32),
            pltpu.VMEM((CH, 8), jnp.float32),
            pltpu.VMEM((CH, 8), jnp.float32),
            pltpu.VMEM((CH,), jnp.float32),
            pltpu.VMEM((128,), jnp.float32),
            pltpu.VMEM((256,), jnp.float32),
            pltpu.VMEM((256,), jnp.float32),
            pltpu.SemaphoreType.DMA,
            pltpu.SemaphoreType.DMA,
        ],
    )
    out = run(pairs_i, pairs_j, packed, a_flat, b_flat, consts)
    return out[:p]


# speedup vs baseline: 275.2491x; 1.3502x over previous
import functools

import jax
import jax.numpy as jnp
from jax import lax
from jax.experimental import pallas as pl
from jax.experimental.pallas import tpu as pltpu
from jax.experimental.pallas import tpu_sc as plsc

NW = 32
CH = 2048
GB = 128
NG = CH // GB
LANES = 16


def _sc_body(ppw, nch, pairs_hbm, packed_hbm, af_hbm, bf_hbm,
             consts_hbm, out_hbm, idx_b0, idx_b1, rows_i0, rows_j0,
             rows_i1, rows_j1, ene_b0, ene_b1, consts_v, a_v, b_v,
             sem_x0, sem_x1, sem_i0, sem_j0, sem_i1, sem_j1,
             sem_o0, sem_o1):
    wid = lax.axis_index("s") * 2 + lax.axis_index("c")
    base = wid * ppw

    pltpu.sync_copy(consts_hbm, consts_v)
    pltpu.sync_copy(af_hbm, a_v)
    pltpu.sync_copy(bf_hbm, b_v)

    ibx = consts_v[pl.ds(0, LANES)]
    iby = consts_v[pl.ds(16, LANES)]
    ibz = consts_v[pl.ds(32, LANES)]
    lbx = consts_v[pl.ds(48, LANES)]
    lby = consts_v[pl.ds(64, LANES)]
    lbz = consts_v[pl.ds(80, LANES)]
    cut = consts_v[pl.ds(96, LANES)]
    zero = jnp.zeros((LANES,), jnp.float32)
    half = zero + 0.5
    one = zero + 1.0

    iota = lax.iota(jnp.int32, LANES)
    col0 = jnp.zeros((LANES,), jnp.int32)
    col1 = col0 + 1
    col2 = col0 + 2
    col3 = col0 + 3

    def bf16r(v):
        b = plsc.bitcast(v, jnp.int32)
        lsb = lax.shift_right_logical(b, 16) & 1
        b = b + lsb + jnp.int32(0x7FFF)
        b = b & jnp.int32(-65536)
        return plsc.bitcast(b, jnp.float32)

    def wrap(d, ib, lb):
        f = bf16r(d) * ib
        f = f + jnp.where(f > half, -one, jnp.where(f < -half, one, zero))
        return bf16r(f) * lb

    def issue_idx(c, idx_b, sem_x):
        row_off = (base + c * CH) // GB
        pltpu.async_copy(pairs_hbm.at[pl.ds(row_off, NG)], idx_b, sem_x)

    def issue_gather(idx_b, sem_x, rows_i, rows_j, sem_i, sem_j):
        pltpu.make_async_copy(pairs_hbm.at[pl.ds(0, NG)], idx_b, sem_x).wait()
        for g in range(NG):
            pltpu.async_copy(packed_hbm.at[idx_b.at[g, 0]],
                             rows_i.at[pl.ds(g * GB, GB)], sem_i)
            pltpu.async_copy(packed_hbm.at[idx_b.at[g, 1]],
                             rows_j.at[pl.ds(g * GB, GB)], sem_j)

    def compute(c, rows_i, rows_j, sem_i, sem_j, ene_b, sem_o):
        dummy = packed_hbm.at[pl.ds(0, CH)]
        pltpu.make_async_copy(dummy, rows_i, sem_i).wait()
        pltpu.make_async_copy(dummy, rows_j, sem_j).wait()

        @pl.when(c >= 2)
        def _():
            pltpu.make_async_copy(ene_b, out_hbm.at[pl.ds(0, CH)],
                                  sem_o).wait()

        @pl.loop(0, CH // LANES)
        def _step(k):
            s = k * LANES
            ridx = iota + s
            xi = plsc.load_gather(rows_i, [ridx, col0])
            yi = plsc.load_gather(rows_i, [ridx, col1])
            zi = plsc.load_gather(rows_i, [ridx, col2])
            ti = plsc.load_gather(rows_i, [ridx, col3])
            xj = plsc.load_gather(rows_j, [ridx, col0])
            yj = plsc.load_gather(rows_j, [ridx, col1])
            zj = plsc.load_gather(rows_j, [ridx, col2])
            tj = plsc.load_gather(rows_j, [ridx, col3])

            dx = wrap(xj - xi, ibx, lbx)
            dy = wrap(yj - yi, iby, lby)
            dz = wrap(zj - zi, ibz, lbz)
            d2 = dx * dx + dy * dy + dz * dz

            bits = plsc.bitcast(d2, jnp.int32)
            seed = jnp.int32(0x5F3759DF) - lax.shift_right_logical(bits, 1)
            y = plsc.bitcast(seed, jnp.float32)
            hd = 0.5 * d2
            y = y * (1.5 - hd * y * y)
            y = y * (1.5 - hd * y * y)
            y = y * (1.5 - hd * y * y)
            r = d2 * y

            tcomb = ti.astype(jnp.int32) * 16 + tj.astype(jnp.int32)
            a = plsc.load_gather(a_v, [tcomb])
            b = plsc.load_gather(b_v, [tcomb])

            x = b * r
            poly = x * x * (1.0 / 3.0) + x + 1.0
            e = a * poly * jnp.exp(-x)
            e = jnp.where(r <= cut, e, zero)
            ene_b[pl.ds(s, LANES)] = e

        pltpu.async_copy(ene_b, out_hbm.at[pl.ds(base + c * CH, CH)], sem_o)

    issue_idx(0, idx_b0, sem_x0)
    issue_idx(1, idx_b1, sem_x1)
    issue_gather(idx_b0, sem_x0, rows_i0, rows_j0, sem_i0, sem_j0)

    @pl.loop(0, nch // 2)
    def _it(it):
        c0 = it * 2
        c1 = c0 + 1
        issue_gather(idx_b1, sem_x1, rows_i1, rows_j1, sem_i1, sem_j1)

        @pl.when(c0 + 2 < nch)
        def _():
            issue_idx(c0 + 2, idx_b0, sem_x0)

        compute(c0, rows_i0, rows_j0, sem_i0, sem_j0, ene_b0, sem_o0)

        @pl.when(c0 + 2 < nch)
        def _():
            issue_gather(idx_b0, sem_x0, rows_i0, rows_j0, sem_i0, sem_j0)
            issue_idx(c1 + 2, idx_b1, sem_x1)

        compute(c1, rows_i1, rows_j1, sem_i1, sem_j1, ene_b1, sem_o1)

    pltpu.make_async_copy(ene_b0, out_hbm.at[pl.ds(0, CH)], sem_o0).wait()
    pltpu.make_async_copy(ene_b1, out_hbm.at[pl.ds(0, CH)], sem_o1).wait()


def kernel(coords, pairs, box, A, B, cutoff, atom_types):
    n = coords.shape[0]
    p = pairs.shape[0]
    nch = -(-p // (NW * CH))
    if nch % 2:
        nch += 1
    ppw = nch * CH
    p_pad = NW * ppw

    pairs_i = pairs[:, 0]
    pairs_j = pairs[:, 1]
    pad = p_pad - p
    if pad:
        zpad = jnp.zeros((pad,), jnp.int32)
        pairs_i = jnp.concatenate([pairs_i, zpad])
        pairs_j = jnp.concatenate([pairs_j, zpad])
    pairs_blk = jnp.stack([pairs_i.reshape(-1, GB),
                           pairs_j.reshape(-1, GB)], axis=1)

    packed = jnp.concatenate(
        [coords.astype(jnp.float32),
         atom_types.astype(jnp.float32).reshape(n, 1),
         jnp.zeros((n, 4), jnp.float32)], axis=1)
    a_flat = A.astype(jnp.float32).reshape(-1)
    b_flat = B.astype(jnp.float32).reshape(-1)

    inv_box = jnp.linalg.inv(box)
    ib = jnp.diagonal(inv_box).astype(jnp.bfloat16).astype(jnp.float32)
    lb = jnp.diagonal(box).astype(jnp.bfloat16).astype(jnp.float32)
    cutf = jnp.asarray(cutoff, jnp.float32)
    vals = jnp.stack([ib[0], ib[1], ib[2], lb[0], lb[1], lb[2],
                      cutf, jnp.float32(0.0)])
    consts = jnp.repeat(vals, LANES)

    mesh = plsc.VectorSubcoreMesh(core_axis_name="c", subcore_axis_name="s")
    run = pl.kernel(
        functools.partial(_sc_body, ppw, nch),
        out_type=jax.ShapeDtypeStruct((p_pad,), jnp.float32),
        mesh=mesh,
        compiler_params=pltpu.CompilerParams(
            needs_layout_passes=False, use_tc_tiling_on_sc=False),
        scratch_types=[
            pltpu.VMEM((NG, 2, GB), jnp.int32),
            pltpu.VMEM((NG, 2, GB), jnp.int32),
            pltpu.VMEM((CH, 8), jnp.float32),
            pltpu.VMEM((CH, 8), jnp.float32),
            pltpu.VMEM((CH, 8), jnp.float32),
            pltpu.VMEM((CH, 8), jnp.float32),
            pltpu.VMEM((CH,), jnp.float32),
            pltpu.VMEM((CH,), jnp.float32),
            pltpu.VMEM((128,), jnp.float32),
            pltpu.VMEM((256,), jnp.float32),
            pltpu.VMEM((256,), jnp.float32),
            pltpu.SemaphoreType.DMA,
            pltpu.SemaphoreType.DMA,
            pltpu.SemaphoreType.DMA,
            pltpu.SemaphoreType.DMA,
            pltpu.SemaphoreType.DMA,
            pltpu.SemaphoreType.DMA,
            pltpu.SemaphoreType.DMA,
            pltpu.SemaphoreType.DMA,
        ],
    )
    out = run(pairs_blk, packed, a_flat, b_flat, consts)
    return out[:p]
